# baseline (speedup 1.0000x reference)
import functools

import jax
import jax.numpy as jnp
from jax import lax
from jax.experimental import pallas as pl
from jax.experimental.pallas import tpu as pltpu
from jax.experimental.pallas import tpu_sc as plsc

DIM = 64
BATCH = 16384
ROWS = 2 * BATCH
NC = 2
NS = 16
NW = NC * NS
BPW = ROWS // NW
RPW = BATCH // NW
NGRP = BPW // 16


def _build_gather():
    mesh = plsc.VectorSubcoreMesh(core_axis_name="c", subcore_axis_name="s")

    @functools.partial(
        pl.kernel,
        mesh=mesh,
        compiler_params=pltpu.CompilerParams(needs_layout_passes=False),
        out_type=jax.ShapeDtypeStruct((BATCH, 2 * DIM), jnp.float32),
        scratch_types=[
            pltpu.VMEM((8, 128), jnp.int32),
            pltpu.SemaphoreType.DMA,
        ],
    )
    def gather_k(idx_hbm, table_hbm, out_hbm, idx_v, sem):
        wid = lax.axis_index("s") * NC + lax.axis_index("c")
        pltpu.sync_copy(idx_hbm.at[wid], idx_v)
        row_base = wid * RPW

        def row_dma(g, j):
            su = lax.shift_right_logical(g, 3)
            lane0 = jnp.bitwise_and(g, 7) * 16
            vv = idx_v[su, pl.ds(lane0, 16)]
            v = vv[j]
            r = row_base + g * 8 + (j >> 1)
            c0 = 64 * (j & 1)
            return pltpu.make_async_copy(
                table_hbm.at[v], out_hbm.at[r, pl.ds(c0, DIM)], sem)

        def fire(g):
            for j in range(16):
                row_dma(g, j).start()

        def drain(g):
            for j in range(16):
                row_dma(g, j).wait()

        pl.loop(0, NGRP)(fire)
        pl.loop(0, NGRP)(drain)

    return gather_k


_gather = _build_gather()

BLK = 1024
GRID = BATCH // BLK


def _mlp_body(x_ref, w1t_ref, b1_ref, wn_ref, ws_ref, bias_ref,
              out_n_ref, out_s_ref):
    x = x_ref[...]
    h = jnp.dot(x, w1t_ref[...], preferred_element_type=jnp.float32)
    h = jnp.maximum(h + b1_ref[...], 0.0)
    n = jnp.sum(h * wn_ref[...], axis=1, keepdims=True) + bias_ref[0]
    s = jnp.sum(h * ws_ref[...], axis=1, keepdims=True) + bias_ref[1]
    out_n_ref[...] = jax.nn.sigmoid(n)
    out_s_ref[...] = jax.nn.sigmoid(s)


def _mlp(x, w1t, b1r, wn, ws, bias2):
    return pl.pallas_call(
        _mlp_body,
        grid=(GRID,),
        in_specs=[
            pl.BlockSpec((BLK, 2 * DIM), lambda i: (i, 0)),
            pl.BlockSpec((2 * DIM, 2 * DIM), lambda i: (0, 0)),
            pl.BlockSpec((1, 2 * DIM), lambda i: (0, 0)),
            pl.BlockSpec((1, 2 * DIM), lambda i: (0, 0)),
            pl.BlockSpec((1, 2 * DIM), lambda i: (0, 0)),
            pl.BlockSpec(memory_space=pltpu.SMEM),
        ],
        out_specs=[
            pl.BlockSpec((BLK, 1), lambda i: (i, 0)),
            pl.BlockSpec((BLK, 1), lambda i: (i, 0)),
        ],
        out_shape=[
            jax.ShapeDtypeStruct((BATCH, 1), jnp.float32),
            jax.ShapeDtypeStruct((BATCH, 1), jnp.float32),
        ],
    )(x, w1t, b1r, wn, ws, bias2)


def kernel(inputs, emb, W1, b1, Wn, bn, Ws, bs):
    idx = inputs.reshape(NW, 8, 128)
    x = _gather(idx, emb)
    bias2 = jnp.concatenate([bn, bs])
    out_n, out_s = _mlp(x, W1.T, b1.reshape(1, 2 * DIM), Wn, Ws, bias2)
    return (out_n, out_s)


# speedup vs baseline: 1.2244x; 1.2244x over previous
import functools

import jax
import jax.numpy as jnp
from jax import lax
from jax.experimental import pallas as pl
from jax.experimental.pallas import tpu as pltpu
from jax.experimental.pallas import tpu_sc as plsc

DIM = 64
BATCH = 16384
ROWS = 2 * BATCH
VOC = 1000000
NC = 2
NS = 16
NW = NC * NS
BPW = ROWS // NW
RPW = BATCH // NW
C = 128
NSTAGE = BPW // C


PK_VB = 2048
PK_GRID = -(-VOC // PK_VB)
VOC2 = PK_GRID * (PK_VB // 2)


def _pack_body(xt_ref, p_ref):
    t = xt_ref[...]
    a = lax.slice(t, (0, 0), (DIM, PK_VB // 2))
    b = lax.slice(t, (0, PK_VB // 2), (DIM, PK_VB))
    p_ref[...] = jnp.concatenate([a.T, b.T], axis=1)


def _pack(embT):
    return pl.pallas_call(
        _pack_body,
        grid=(PK_GRID,),
        in_specs=[pl.BlockSpec((DIM, PK_VB), lambda i: (0, i))],
        out_specs=pl.BlockSpec((PK_VB // 2, 2 * DIM), lambda i: (i, 0)),
        out_shape=jax.ShapeDtypeStruct((VOC2, 2 * DIM), jnp.float32),
    )(embT)


def _build_gather():
    mesh = plsc.VectorSubcoreMesh(core_axis_name="c", subcore_axis_name="s")

    @functools.partial(
        pl.kernel,
        mesh=mesh,
        compiler_params=pltpu.CompilerParams(needs_layout_passes=False),
        out_type=jax.ShapeDtypeStruct((BATCH, 2 * DIM), jnp.float32),
        scratch_types=[
            pltpu.VMEM((8, 128), jnp.int32),
            pltpu.VMEM((BPW,), jnp.int32),
            pltpu.VMEM((BPW,), jnp.int32),
            pltpu.VMEM((C, 2 * DIM), jnp.float32),
            pltpu.VMEM((C, 2 * DIM), jnp.float32),
            pltpu.VMEM((RPW, 2 * DIM), jnp.float32),
            pltpu.SemaphoreType.DMA,
            pltpu.SemaphoreType.DMA,
        ],
    )
    def gather_k(idx_hbm, p_hbm, out_hbm,
                 idx_v, idx1_v, pk_v, buf0, buf1, out_v, sem0, sem1):
        wid = lax.axis_index("s") * NC + lax.axis_index("c")
        pltpu.sync_copy(idx_hbm.at[wid], idx_v)
        for su in range(8):
            for l in range(8):
                v = idx_v[su, pl.ds(16 * l, 16)]
                idx1_v[pl.ds(su * 128 + 16 * l, 16)] = v
                pk = (lax.shift_right_logical(v, 11) * 1024
                      + jnp.bitwise_and(v, 1023))
                pk_v[pl.ds(su * 128 + 16 * l, 16)] = pk

        bufs = (buf0, buf1)
        sems = (sem0, sem1)

        def stage_copy(s, b):
            off = pl.multiple_of(s * C, 8)
            return pltpu.make_async_copy(
                p_hbm.at[pk_v.at[pl.ds(off, C)]], bufs[b], sems[b])

        stage_copy(0, 0).start()
        lane = lax.iota(jnp.int32, 16)

        def extract(s, buf):
            r0 = s * (C // 2)

            def per_j(j):
                vv = plsc.load_gather(
                    idx1_v, [jnp.full((16,), s * C + j, jnp.int32)])
                hv = jnp.bitwise_and(lax.shift_right_logical(vv, 10), 1) * 64
                jv = jnp.full((16,), j, jnp.int32)
                r = r0 + (j >> 1)
                c0 = 64 * jnp.bitwise_and(j, 1)
                for k in range(4):
                    x = plsc.load_gather(buf, [jv, hv + lane + 16 * k])
                    out_v[r, pl.ds(c0 + 16 * k, 16)] = x

            pl.loop(0, C)(per_j)

        def body(s0):
            for b in range(2):
                s = s0 + b

                @pl.when(s + 1 < NSTAGE)
                def _():
                    stage_copy(s + 1, 1 - b).start()

                stage_copy(s, b).wait()
                extract(s, bufs[b])

        pl.loop(0, NSTAGE, step=2)(body)
        pltpu.sync_copy(out_v, out_hbm.at[pl.ds(wid * RPW, RPW)])

    return gather_k


_gather = _build_gather()


BLK = 1024
GRID = BATCH // BLK


def _mlp_body(x_ref, w1t_ref, b1_ref, wn_ref, ws_ref, bias_ref,
              out_n_ref, out_s_ref):
    x = x_ref[...]
    h = jnp.dot(x, w1t_ref[...], preferred_element_type=jnp.float32)
    h = jnp.maximum(h + b1_ref[...], 0.0)
    n = jnp.sum(h * wn_ref[...], axis=1, keepdims=True) + bias_ref[0]
    s = jnp.sum(h * ws_ref[...], axis=1, keepdims=True) + bias_ref[1]
    out_n_ref[...] = jax.nn.sigmoid(n)
    out_s_ref[...] = jax.nn.sigmoid(s)


def _mlp(x, w1t, b1r, wn, ws, bias2):
    return pl.pallas_call(
        _mlp_body,
        grid=(GRID,),
        in_specs=[
            pl.BlockSpec((BLK, 2 * DIM), lambda i: (i, 0)),
            pl.BlockSpec((2 * DIM, 2 * DIM), lambda i: (0, 0)),
            pl.BlockSpec((1, 2 * DIM), lambda i: (0, 0)),
            pl.BlockSpec((1, 2 * DIM), lambda i: (0, 0)),
            pl.BlockSpec((1, 2 * DIM), lambda i: (0, 0)),
            pl.BlockSpec(memory_space=pltpu.SMEM),
        ],
        out_specs=[
            pl.BlockSpec((BLK, 1), lambda i: (i, 0)),
            pl.BlockSpec((BLK, 1), lambda i: (i, 0)),
        ],
        out_shape=[
            jax.ShapeDtypeStruct((BATCH, 1), jnp.float32),
            jax.ShapeDtypeStruct((BATCH, 1), jnp.float32),
        ],
    )(x, w1t, b1r, wn, ws, bias2)


def kernel(inputs, emb, W1, b1, Wn, bn, Ws, bs):
    p = _pack(emb.T)
    idx = inputs.reshape(NW, 8, 128)
    x = _gather(idx, p)
    bias2 = jnp.concatenate([bn, bs])
    out_n, out_s = _mlp(x, W1.T, b1.reshape(1, 2 * DIM), Wn, Ws, bias2)
    return (out_n, out_s)


# speedup vs baseline: 1.5748x; 1.2862x over previous
import functools

import jax
import jax.numpy as jnp
from jax import lax
from jax.experimental import pallas as pl
from jax.experimental.pallas import tpu as pltpu
from jax.experimental.pallas import tpu_sc as plsc

DIM = 64
BATCH = 16384
ROWS = 2 * BATCH
VOC = 1000000
NC = 2
NS = 16
NW = NC * NS
BPW = ROWS // NW
RPW = BATCH // NW
C = 128
NSTAGE = BPW // C


PK_VB = 4096
PK_SH = 12
PK_H = PK_VB // 2
PK_GRID = -(-VOC // PK_VB)
VOC2 = PK_GRID * PK_H


def _pack_body(xt_ref, p_ref):
    t = xt_ref[...]
    a = lax.slice(t, (0, 0), (DIM, PK_H))
    b = lax.slice(t, (0, PK_H), (DIM, PK_VB))
    p_ref[:, 0:DIM] = a.T
    p_ref[:, DIM:2 * DIM] = b.T


def _pack(embT):
    return pl.pallas_call(
        _pack_body,
        grid=(PK_GRID,),
        in_specs=[pl.BlockSpec((DIM, PK_VB), lambda i: (0, i))],
        out_specs=pl.BlockSpec((PK_H, 2 * DIM), lambda i: (i, 0)),
        out_shape=jax.ShapeDtypeStruct((VOC2, 2 * DIM), jnp.float32),
    )(embT)


def _build_gather():
    mesh = plsc.VectorSubcoreMesh(core_axis_name="c", subcore_axis_name="s")

    @functools.partial(
        pl.kernel,
        mesh=mesh,
        compiler_params=pltpu.CompilerParams(needs_layout_passes=False),
        out_type=jax.ShapeDtypeStruct((BATCH, 2 * DIM), jnp.float32),
        scratch_types=[
            pltpu.VMEM((8, 128), jnp.int32),
            pltpu.VMEM((BPW,), jnp.int32),
            pltpu.VMEM((BPW,), jnp.int32),
            pltpu.VMEM((C, 2 * DIM), jnp.float32),
            pltpu.VMEM((C, 2 * DIM), jnp.float32),
            pltpu.VMEM((RPW, 2 * DIM), jnp.float32),
            pltpu.SemaphoreType.DMA,
            pltpu.SemaphoreType.DMA,
        ],
    )
    def gather_k(idx_hbm, p_hbm, out_hbm,
                 idx_v, idx1_v, pk_v, buf0, buf1, out_v, sem0, sem1):
        wid = lax.axis_index("s") * NC + lax.axis_index("c")
        pltpu.sync_copy(idx_hbm.at[wid], idx_v)
        for su in range(8):
            for l in range(8):
                v = idx_v[su, pl.ds(16 * l, 16)]
                idx1_v[pl.ds(su * 128 + 16 * l, 16)] = v
                pk = (lax.shift_right_logical(v, PK_SH) * PK_H
                      + jnp.bitwise_and(v, PK_H - 1))
                pk_v[pl.ds(su * 128 + 16 * l, 16)] = pk

        bufs = (buf0, buf1)
        sems = (sem0, sem1)

        def stage_copy(s, b):
            off = pl.multiple_of(s * C, 8)
            return pltpu.make_async_copy(
                p_hbm.at[pk_v.at[pl.ds(off, C)]], bufs[b], sems[b])

        stage_copy(0, 0).start()
        lane = lax.iota(jnp.int32, 16)

        def extract(s, buf):
            r0 = s * (C // 2)

            def per_j(j):
                vv = plsc.load_gather(
                    idx1_v, [jnp.full((16,), s * C + j, jnp.int32)])
                hv = jnp.bitwise_and(
                    lax.shift_right_logical(vv, PK_SH - 1), 1) * 64
                jv = jnp.full((16,), j, jnp.int32)
                r = r0 + (j >> 1)
                c0 = 64 * jnp.bitwise_and(j, 1)
                for k in range(4):
                    x = plsc.load_gather(buf, [jv, hv + lane + 16 * k])
                    out_v[r, pl.ds(c0 + 16 * k, 16)] = x

            pl.loop(0, C)(per_j)

        def body(s0):
            for b in range(2):
                s = s0 + b

                @pl.when(s + 1 < NSTAGE)
                def _():
                    stage_copy(s + 1, 1 - b).start()

                stage_copy(s, b).wait()
                extract(s, bufs[b])

        pl.loop(0, NSTAGE, step=2)(body)
        pltpu.sync_copy(out_v, out_hbm.at[pl.ds(wid * RPW, RPW)])

    return gather_k


_gather = _build_gather()


BLK = 1024
GRID = BATCH // BLK


def _mlp_body(x_ref, w1t_ref, b1_ref, wn_ref, ws_ref, bias_ref,
              out_n_ref, out_s_ref):
    x = x_ref[...]
    h = jnp.dot(x, w1t_ref[...], preferred_element_type=jnp.float32)
    h = jnp.maximum(h + b1_ref[...], 0.0)
    n = jnp.sum(h * wn_ref[...], axis=1, keepdims=True) + bias_ref[0]
    s = jnp.sum(h * ws_ref[...], axis=1, keepdims=True) + bias_ref[1]
    out_n_ref[...] = jax.nn.sigmoid(n)
    out_s_ref[...] = jax.nn.sigmoid(s)


def _mlp(x, w1t, b1r, wn, ws, bias2):
    return pl.pallas_call(
        _mlp_body,
        grid=(GRID,),
        in_specs=[
            pl.BlockSpec((BLK, 2 * DIM), lambda i: (i, 0)),
            pl.BlockSpec((2 * DIM, 2 * DIM), lambda i: (0, 0)),
            pl.BlockSpec((1, 2 * DIM), lambda i: (0, 0)),
            pl.BlockSpec((1, 2 * DIM), lambda i: (0, 0)),
            pl.BlockSpec((1, 2 * DIM), lambda i: (0, 0)),
            pl.BlockSpec(memory_space=pltpu.SMEM),
        ],
        out_specs=[
            pl.BlockSpec((BLK, 1), lambda i: (i, 0)),
            pl.BlockSpec((BLK, 1), lambda i: (i, 0)),
        ],
        out_shape=[
            jax.ShapeDtypeStruct((BATCH, 1), jnp.float32),
            jax.ShapeDtypeStruct((BATCH, 1), jnp.float32),
        ],
    )(x, w1t, b1r, wn, ws, bias2)


def kernel(inputs, emb, W1, b1, Wn, bn, Ws, bs):
    p = _pack(emb.T)
    idx = inputs.reshape(NW, 8, 128)
    x = _gather(idx, p)
    bias2 = jnp.concatenate([bn, bs])
    out_n, out_s = _mlp(x, W1.T, b1.reshape(1, 2 * DIM), Wn, Ws, bias2)
    return (out_n, out_s)


# speedup vs baseline: 1.8862x; 1.1978x over previous
import functools

import jax
import jax.numpy as jnp
from jax import lax
from jax.experimental import pallas as pl
from jax.experimental.pallas import tpu as pltpu
from jax.experimental.pallas import tpu_sc as plsc

DIM = 64
BATCH = 16384
ROWS = 2 * BATCH
VOC = 1000000
NC = 2
NS = 16
NW = NC * NS
BPW = ROWS // NW
RPW = BATCH // NW
C = 128
NSTAGE = BPW // C


PK_VB = 8192
PK_SH = 13
PK_H = PK_VB // 2
PK_GRID = -(-VOC // PK_VB)
VOC2 = PK_GRID * PK_H


def _pack_body(xt_ref, p_ref):
    t = xt_ref[...]
    a = lax.slice(t, (0, 0), (DIM, PK_H))
    b = lax.slice(t, (0, PK_H), (DIM, PK_VB))
    p_ref[:, 0:DIM] = a.T
    p_ref[:, DIM:2 * DIM] = b.T


def _pack(embT):
    return pl.pallas_call(
        _pack_body,
        grid=(PK_GRID,),
        in_specs=[pl.BlockSpec((DIM, PK_VB), lambda i: (0, i))],
        out_specs=pl.BlockSpec((PK_H, 2 * DIM), lambda i: (i, 0)),
        out_shape=jax.ShapeDtypeStruct((VOC2, 2 * DIM), jnp.float32),
    )(embT)


def _build_gather():
    mesh = plsc.VectorSubcoreMesh(core_axis_name="c", subcore_axis_name="s")

    @functools.partial(
        pl.kernel,
        mesh=mesh,
        compiler_params=pltpu.CompilerParams(needs_layout_passes=False),
        out_type=jax.ShapeDtypeStruct((BATCH, 2 * DIM), jnp.float32),
        scratch_types=[
            pltpu.VMEM((8, 128), jnp.int32),
            pltpu.VMEM((BPW,), jnp.int32),
            pltpu.VMEM((BPW,), jnp.int32),
            pltpu.VMEM((C, 2 * DIM), jnp.float32),
            pltpu.VMEM((C, 2 * DIM), jnp.float32),
            pltpu.VMEM((RPW, 2 * DIM), jnp.float32),
            pltpu.SemaphoreType.DMA,
            pltpu.SemaphoreType.DMA,
        ],
    )
    def gather_k(idx_hbm, p_hbm, out_hbm,
                 idx_v, idx1_v, pk_v, buf0, buf1, out_v, sem0, sem1):
        wid = lax.axis_index("s") * NC + lax.axis_index("c")
        pltpu.sync_copy(idx_hbm.at[wid], idx_v)
        for su in range(8):
            for l in range(8):
                v = idx_v[su, pl.ds(16 * l, 16)]
                idx1_v[pl.ds(su * 128 + 16 * l, 16)] = v
                pk = (lax.shift_right_logical(v, PK_SH) * PK_H
                      + jnp.bitwise_and(v, PK_H - 1))
                pk_v[pl.ds(su * 128 + 16 * l, 16)] = pk

        bufs = (buf0, buf1)
        sems = (sem0, sem1)

        def stage_copy(s, b):
            off = pl.multiple_of(s * C, 8)
            return pltpu.make_async_copy(
                p_hbm.at[pk_v.at[pl.ds(off, C)]], bufs[b], sems[b])

        stage_copy(0, 0).start()
        lane = lax.iota(jnp.int32, 16)

        def extract(s, buf):
            r0 = s * (C // 2)

            def per_j(j):
                vv = plsc.load_gather(
                    idx1_v, [jnp.full((16,), s * C + j, jnp.int32)])
                hv = jnp.bitwise_and(
                    lax.shift_right_logical(vv, PK_SH - 1), 1) * 64
                jv = jnp.full((16,), j, jnp.int32)
                r = r0 + (j >> 1)
                c0 = 64 * jnp.bitwise_and(j, 1)
                for k in range(4):
                    x = plsc.load_gather(buf, [jv, hv + lane + 16 * k])
                    out_v[r, pl.ds(c0 + 16 * k, 16)] = x

            pl.loop(0, C)(per_j)

        def body(s0):
            for b in range(2):
                s = s0 + b

                @pl.when(s + 1 < NSTAGE)
                def _():
                    stage_copy(s + 1, 1 - b).start()

                stage_copy(s, b).wait()
                extract(s, bufs[b])

        pl.loop(0, NSTAGE, step=2)(body)
        pltpu.sync_copy(out_v, out_hbm.at[pl.ds(wid * RPW, RPW)])

    return gather_k


_gather = _build_gather()


BLK = 1024
GRID = BATCH // BLK


def _mlp_body(x_ref, w1t_ref, b1_ref, wn_ref, ws_ref, bias_ref,
              out_n_ref, out_s_ref):
    x = x_ref[...]
    h = jnp.dot(x, w1t_ref[...], preferred_element_type=jnp.float32)
    h = jnp.maximum(h + b1_ref[...], 0.0)
    n = jnp.sum(h * wn_ref[...], axis=1, keepdims=True) + bias_ref[0]
    s = jnp.sum(h * ws_ref[...], axis=1, keepdims=True) + bias_ref[1]
    out_n_ref[...] = jax.nn.sigmoid(n)
    out_s_ref[...] = jax.nn.sigmoid(s)


def _mlp(x, w1t, b1r, wn, ws, bias2):
    return pl.pallas_call(
        _mlp_body,
        grid=(GRID,),
        in_specs=[
            pl.BlockSpec((BLK, 2 * DIM), lambda i: (i, 0)),
            pl.BlockSpec((2 * DIM, 2 * DIM), lambda i: (0, 0)),
            pl.BlockSpec((1, 2 * DIM), lambda i: (0, 0)),
            pl.BlockSpec((1, 2 * DIM), lambda i: (0, 0)),
            pl.BlockSpec((1, 2 * DIM), lambda i: (0, 0)),
            pl.BlockSpec(memory_space=pltpu.SMEM),
        ],
        out_specs=[
            pl.BlockSpec((BLK, 1), lambda i: (i, 0)),
            pl.BlockSpec((BLK, 1), lambda i: (i, 0)),
        ],
        out_shape=[
            jax.ShapeDtypeStruct((BATCH, 1), jnp.float32),
            jax.ShapeDtypeStruct((BATCH, 1), jnp.float32),
        ],
    )(x, w1t, b1r, wn, ws, bias2)


def kernel(inputs, emb, W1, b1, Wn, bn, Ws, bs):
    p = _pack(emb.T)
    idx = inputs.reshape(NW, 8, 128)
    x = _gather(idx, p)
    bias2 = jnp.concatenate([bn, bs])
    out_n, out_s = _mlp(x, W1.T, b1.reshape(1, 2 * DIM), Wn, Ws, bias2)
    return (out_n, out_s)


# speedup vs baseline: 2.2058x; 1.1694x over previous
import functools

import jax
import jax.numpy as jnp
from jax import lax
from jax.experimental import pallas as pl
from jax.experimental.pallas import tpu as pltpu
from jax.experimental.pallas import tpu_sc as plsc

DIM = 64
BATCH = 16384
ROWS = 2 * BATCH
VOC = 1000000
NC = 2
NS = 16
NW = NC * NS
BPW = ROWS // NW
RPW = BATCH // NW
C = 128
NSTAGE = BPW // C


PK_VB = 16384
PK_SH = 14
PK_H = PK_VB // 2
PK_GRID = -(-VOC // PK_VB)
VOC2 = PK_GRID * PK_H


def _pack_body(xt_ref, p_ref):
    t = xt_ref[...]
    a = lax.slice(t, (0, 0), (DIM, PK_H))
    b = lax.slice(t, (0, PK_H), (DIM, PK_VB))
    p_ref[:, 0:DIM] = a.T
    p_ref[:, DIM:2 * DIM] = b.T


def _pack(embT):
    return pl.pallas_call(
        _pack_body,
        grid=(PK_GRID,),
        in_specs=[pl.BlockSpec((DIM, PK_VB), lambda i: (0, i))],
        out_specs=pl.BlockSpec((PK_H, 2 * DIM), lambda i: (i, 0)),
        out_shape=jax.ShapeDtypeStruct((VOC2, 2 * DIM), jnp.float32),
    )(embT)


def _build_gather():
    mesh = plsc.VectorSubcoreMesh(core_axis_name="c", subcore_axis_name="s")

    @functools.partial(
        pl.kernel,
        mesh=mesh,
        compiler_params=pltpu.CompilerParams(needs_layout_passes=False),
        out_type=jax.ShapeDtypeStruct((BATCH, 2 * DIM), jnp.float32),
        scratch_types=[
            pltpu.VMEM((BPW,), jnp.int32),
            pltpu.VMEM((BPW,), jnp.int32),
            pltpu.VMEM((C, 2 * DIM), jnp.float32),
            pltpu.VMEM((C, 2 * DIM), jnp.float32),
            pltpu.VMEM((RPW, 2 * DIM), jnp.float32),
            pltpu.SemaphoreType.DMA,
            pltpu.SemaphoreType.DMA,
        ],
    )
    def gather_k(idx_hbm, p_hbm, out_hbm,
                 idx1_v, pk_v, buf0, buf1, out_v, sem0, sem1):
        wid = lax.axis_index("s") * NC + lax.axis_index("c")
        b0 = wid * RPW
        pltpu.sync_copy(idx_hbm.at[0, pl.ds(b0, RPW)], idx1_v.at[pl.ds(0, RPW)])
        pltpu.sync_copy(idx_hbm.at[1, pl.ds(b0, RPW)],
                        idx1_v.at[pl.ds(RPW, RPW)])
        for g in range(BPW // 16):
            v = idx1_v[pl.ds(16 * g, 16)]
            pk = (lax.shift_right_logical(v, PK_SH) * PK_H
                  + jnp.bitwise_and(v, PK_H - 1))
            pk_v[pl.ds(16 * g, 16)] = pk

        bufs = (buf0, buf1)
        sems = (sem0, sem1)

        def stage_copy(s, b):
            off = pl.multiple_of(s * C, 8)
            return pltpu.make_async_copy(
                p_hbm.at[pk_v.at[pl.ds(off, C)]], bufs[b], sems[b])

        stage_copy(0, 0).start()
        lane = lax.iota(jnp.int32, 16)

        def extract(s, buf):
            r0 = jnp.bitwise_and(s, 3) * C
            c0 = lax.shift_right_logical(s, 2) * 64

            def per_j(j):
                vv = plsc.load_gather(
                    idx1_v, [jnp.full((16,), s * C + j, jnp.int32)])
                hv = jnp.bitwise_and(
                    lax.shift_right_logical(vv, PK_SH - 1), 1) * 64
                jv = jnp.full((16,), j, jnp.int32)
                r = r0 + j
                for k in range(4):
                    x = plsc.load_gather(buf, [jv, hv + lane + 16 * k])
                    out_v[r, pl.ds(c0 + 16 * k, 16)] = x

            pl.loop(0, C)(per_j)

        def body(s0):
            for b in range(2):
                s = s0 + b

                @pl.when(s + 1 < NSTAGE)
                def _():
                    stage_copy(s + 1, 1 - b).start()

                stage_copy(s, b).wait()
                extract(s, bufs[b])

        pl.loop(0, NSTAGE, step=2)(body)
        pltpu.sync_copy(out_v, out_hbm.at[pl.ds(wid * RPW, RPW)])

    return gather_k


_gather = _build_gather()


BLK = 2048
GRID = BATCH // BLK


def _mlp_body(x_ref, w1t_ref, b1_ref, wn_ref, ws_ref, bias_ref,
              out_n_ref, out_s_ref):
    x = x_ref[...]
    h = jnp.dot(x, w1t_ref[...], preferred_element_type=jnp.float32)
    h = jnp.maximum(h + b1_ref[...], 0.0)
    n = jnp.sum(h * wn_ref[...], axis=1, keepdims=True) + bias_ref[0]
    s = jnp.sum(h * ws_ref[...], axis=1, keepdims=True) + bias_ref[1]
    out_n_ref[...] = jax.nn.sigmoid(n)
    out_s_ref[...] = jax.nn.sigmoid(s)


def _mlp(x, w1t, b1r, wn, ws, bias2):
    return pl.pallas_call(
        _mlp_body,
        grid=(GRID,),
        in_specs=[
            pl.BlockSpec((BLK, 2 * DIM), lambda i: (i, 0)),
            pl.BlockSpec((2 * DIM, 2 * DIM), lambda i: (0, 0)),
            pl.BlockSpec((1, 2 * DIM), lambda i: (0, 0)),
            pl.BlockSpec((1, 2 * DIM), lambda i: (0, 0)),
            pl.BlockSpec((1, 2 * DIM), lambda i: (0, 0)),
            pl.BlockSpec(memory_space=pltpu.SMEM),
        ],
        out_specs=[
            pl.BlockSpec((BLK, 1), lambda i: (i, 0)),
            pl.BlockSpec((BLK, 1), lambda i: (i, 0)),
        ],
        out_shape=[
            jax.ShapeDtypeStruct((BATCH, 1), jnp.float32),
            jax.ShapeDtypeStruct((BATCH, 1), jnp.float32),
        ],
    )(x, w1t, b1r, wn, ws, bias2)


def kernel(inputs, emb, W1, b1, Wn, bn, Ws, bs):
    p = _pack(emb.T)
    x = _gather(inputs.T, p)
    bias2 = jnp.concatenate([bn, bs])
    out_n, out_s = _mlp(x, W1.T, b1.reshape(1, 2 * DIM), Wn, Ws, bias2)
    return (out_n, out_s)


# speedup vs baseline: 2.7373x; 1.2409x over previous
import functools

import jax
import jax.numpy as jnp
from jax import lax
from jax.experimental import pallas as pl
from jax.experimental.pallas import tpu as pltpu
from jax.experimental.pallas import tpu_sc as plsc

DIM = 64
BATCH = 16384
ROWS = 2 * BATCH
VOC = 1000000
NC = 2
NS = 16
NW = NC * NS
BPW = ROWS // NW
RPW = BATCH // NW
C = 128
NSTAGE = BPW // C


PK_VB = 16384
PK_SH = 14
PK_H = PK_VB // 2
PK_GRID = -(-VOC // PK_VB)
VOC2 = PK_GRID * PK_H


def _pack_body(xt_ref, p_ref):
    t = xt_ref[...]
    a = lax.slice(t, (0, 0), (DIM, PK_H))
    b = lax.slice(t, (0, PK_H), (DIM, PK_VB))
    p_ref[...] = jnp.concatenate([a, b], axis=0).T


def _pack(embT):
    return pl.pallas_call(
        _pack_body,
        grid=(PK_GRID,),
        in_specs=[pl.BlockSpec((DIM, PK_VB), lambda i: (0, i))],
        out_specs=pl.BlockSpec((PK_H, 2 * DIM), lambda i: (i, 0)),
        out_shape=jax.ShapeDtypeStruct((VOC2, 2 * DIM), jnp.float32),
    )(embT)


def _build_gather():
    mesh = plsc.VectorSubcoreMesh(core_axis_name="c", subcore_axis_name="s")

    @functools.partial(
        pl.kernel,
        mesh=mesh,
        compiler_params=pltpu.CompilerParams(needs_layout_passes=False),
        out_type=jax.ShapeDtypeStruct((BATCH, 2 * DIM), jnp.float32),
        scratch_types=[
            pltpu.VMEM((BPW,), jnp.int32),
            pltpu.VMEM((BPW,), jnp.int32),
            pltpu.VMEM((C, 2 * DIM), jnp.float32),
            pltpu.VMEM((C, 2 * DIM), jnp.float32),
            pltpu.VMEM((RPW, 2 * DIM), jnp.float32),
            pltpu.SemaphoreType.DMA,
            pltpu.SemaphoreType.DMA,
        ],
    )
    def gather_k(idx_hbm, p_hbm, out_hbm,
                 idx1_v, pk_v, buf0, buf1, out_v, sem0, sem1):
        wid = lax.axis_index("s") * NC + lax.axis_index("c")
        b0 = wid * RPW
        pltpu.sync_copy(idx_hbm.at[0, pl.ds(b0, RPW)], idx1_v.at[pl.ds(0, RPW)])
        pltpu.sync_copy(idx_hbm.at[1, pl.ds(b0, RPW)],
                        idx1_v.at[pl.ds(RPW, RPW)])
        for g in range(BPW // 16):
            v = idx1_v[pl.ds(16 * g, 16)]
            pk = (lax.shift_right_logical(v, PK_SH) * PK_H
                  + jnp.bitwise_and(v, PK_H - 1))
            pk_v[pl.ds(16 * g, 16)] = pk

        bufs = (buf0, buf1)
        sems = (sem0, sem1)

        def stage_copy(s, b):
            off = pl.multiple_of(s * C, 8)
            return pltpu.make_async_copy(
                p_hbm.at[pk_v.at[pl.ds(off, C)]], bufs[b], sems[b])

        stage_copy(0, 0).start()
        lane = lax.iota(jnp.int32, 16)

        def extract(s, buf):
            r0 = jnp.bitwise_and(s, 3) * C
            c0 = lax.shift_right_logical(s, 2) * 64

            def per_j(j):
                vv = plsc.load_gather(
                    idx1_v, [jnp.full((16,), s * C + j, jnp.int32)])
                hv = jnp.bitwise_and(
                    lax.shift_right_logical(vv, PK_SH - 1), 1) * 64
                jv = jnp.full((16,), j, jnp.int32)
                r = r0 + j
                for k in range(4):
                    x = plsc.load_gather(buf, [jv, hv + lane + 16 * k])
                    out_v[r, pl.ds(c0 + 16 * k, 16)] = x

            pl.loop(0, C)(per_j)

        def body(s0):
            for b in range(2):
                s = s0 + b

                @pl.when(s + 1 < NSTAGE)
                def _():
                    stage_copy(s + 1, 1 - b).start()

                stage_copy(s, b).wait()
                extract(s, bufs[b])

        pl.loop(0, NSTAGE, step=2)(body)
        pltpu.sync_copy(out_v, out_hbm.at[pl.ds(wid * RPW, RPW)])

    return gather_k


_gather = _build_gather()


BLK = 2048
GRID = BATCH // BLK


def _mlp_body(x_ref, w1t_ref, b1_ref, wn_ref, ws_ref, bias_ref,
              out_n_ref, out_s_ref):
    x = x_ref[...]
    h = jnp.dot(x, w1t_ref[...], preferred_element_type=jnp.float32)
    h = jnp.maximum(h + b1_ref[...], 0.0)
    n = jnp.sum(h * wn_ref[...], axis=1, keepdims=True) + bias_ref[0]
    s = jnp.sum(h * ws_ref[...], axis=1, keepdims=True) + bias_ref[1]
    out_n_ref[...] = jax.nn.sigmoid(n)
    out_s_ref[...] = jax.nn.sigmoid(s)


def _mlp(x, w1t, b1r, wn, ws, bias2):
    return pl.pallas_call(
        _mlp_body,
        grid=(GRID,),
        in_specs=[
            pl.BlockSpec((BLK, 2 * DIM), lambda i: (i, 0)),
            pl.BlockSpec((2 * DIM, 2 * DIM), lambda i: (0, 0)),
            pl.BlockSpec((1, 2 * DIM), lambda i: (0, 0)),
            pl.BlockSpec((1, 2 * DIM), lambda i: (0, 0)),
            pl.BlockSpec((1, 2 * DIM), lambda i: (0, 0)),
            pl.BlockSpec(memory_space=pltpu.SMEM),
        ],
        out_specs=[
            pl.BlockSpec((BLK, 1), lambda i: (i, 0)),
            pl.BlockSpec((BLK, 1), lambda i: (i, 0)),
        ],
        out_shape=[
            jax.ShapeDtypeStruct((BATCH, 1), jnp.float32),
            jax.ShapeDtypeStruct((BATCH, 1), jnp.float32),
        ],
    )(x, w1t, b1r, wn, ws, bias2)


def kernel(inputs, emb, W1, b1, Wn, bn, Ws, bs):
    p = _pack(emb.T)
    x = _gather(inputs.T, p)
    bias2 = jnp.concatenate([bn, bs])
    out_n, out_s = _mlp(x, W1.T, b1.reshape(1, 2 * DIM), Wn, Ws, bias2)
    return (out_n, out_s)


# speedup vs baseline: 2.8929x; 1.0568x over previous
import functools

import jax
import jax.numpy as jnp
from jax import lax
from jax.experimental import pallas as pl
from jax.experimental.pallas import tpu as pltpu
from jax.experimental.pallas import tpu_sc as plsc

DIM = 64
BATCH = 16384
ROWS = 2 * BATCH
VOC = 1000000
NC = 2
NS = 16
NW = NC * NS
BPW = ROWS // NW
RPW = BATCH // NW
C = 128
NSTAGE = BPW // C


PK_VB = 16384
PK_SH = 14
PK_H = PK_VB // 2
PK_GRID = -(-VOC // PK_VB)
VOC2 = PK_GRID * PK_H


def _pack_body(xt_ref, p_ref):
    t = xt_ref[...]
    a = lax.slice(t, (0, 0), (DIM, PK_H))
    b = lax.slice(t, (0, PK_H), (DIM, PK_VB))
    p_ref[...] = jnp.concatenate([a, b], axis=0).T


def _pack(embT):
    return pl.pallas_call(
        _pack_body,
        grid=(PK_GRID,),
        in_specs=[pl.BlockSpec((DIM, PK_VB), lambda i: (0, i))],
        out_specs=pl.BlockSpec((PK_H, 2 * DIM), lambda i: (i, 0)),
        out_shape=jax.ShapeDtypeStruct((VOC2, 2 * DIM), jnp.float32),
    )(embT)


def _build_gather():
    mesh = plsc.VectorSubcoreMesh(core_axis_name="c", subcore_axis_name="s")

    @functools.partial(
        pl.kernel,
        mesh=mesh,
        compiler_params=pltpu.CompilerParams(needs_layout_passes=False),
        out_type=jax.ShapeDtypeStruct((BATCH, 2 * DIM), jnp.float32),
        scratch_types=[
            pltpu.VMEM((BPW,), jnp.int32),
            pltpu.VMEM((BPW,), jnp.int32),
            pltpu.VMEM((C, 2 * DIM), jnp.float32),
            pltpu.VMEM((C, 2 * DIM), jnp.float32),
            pltpu.VMEM((RPW, 2 * DIM), jnp.float32),
            pltpu.SemaphoreType.DMA,
            pltpu.SemaphoreType.DMA,
        ],
    )
    def gather_k(idx_hbm, p_hbm, out_hbm,
                 idx1_v, pk_v, buf0, buf1, out_v, sem0, sem1):
        wid = lax.axis_index("s") * NC + lax.axis_index("c")
        b0 = wid * RPW
        pltpu.sync_copy(idx_hbm.at[0, pl.ds(b0, RPW)], idx1_v.at[pl.ds(0, RPW)])
        pltpu.sync_copy(idx_hbm.at[1, pl.ds(b0, RPW)],
                        idx1_v.at[pl.ds(RPW, RPW)])
        for g in range(BPW // 16):
            v = idx1_v[pl.ds(16 * g, 16)]
            pk = (lax.shift_right_logical(v, PK_SH) * PK_H
                  + jnp.bitwise_and(v, PK_H - 1))
            pk_v[pl.ds(16 * g, 16)] = pk

        bufs = (buf0, buf1)
        sems = (sem0, sem1)

        def stage_copy(s, b):
            off = pl.multiple_of(s * C, 8)
            return pltpu.make_async_copy(
                p_hbm.at[pk_v.at[pl.ds(off, C)]], bufs[b], sems[b])

        stage_copy(0, 0).start()
        lane = lax.iota(jnp.int32, 16)

        def extract(s, buf):
            r0 = jnp.bitwise_and(s, 3) * C
            c0 = lax.shift_right_logical(s, 2) * 64

            def per_j(j):
                vv = plsc.load_gather(
                    idx1_v, [jnp.full((16,), s * C + j, jnp.int32)])
                hv = jnp.bitwise_and(
                    lax.shift_right_logical(vv, PK_SH - 1), 1) * 64
                jv = jnp.full((16,), j, jnp.int32)
                r = r0 + j
                for k in range(4):
                    x = plsc.load_gather(buf, [jv, hv + lane + 16 * k])
                    out_v[r, pl.ds(c0 + 16 * k, 16)] = x

            pl.loop(0, C)(per_j)

        def body(s0):
            for b in range(2):
                s = s0 + b

                @pl.when(s + 1 < NSTAGE)
                def _():
                    stage_copy(s + 1, 1 - b).start()

                stage_copy(s, b).wait()
                extract(s, bufs[b])

        pl.loop(0, NSTAGE, step=2)(body)
        pltpu.sync_copy(out_v, out_hbm.at[pl.ds(wid * RPW, RPW)])

    return gather_k


_gather = _build_gather()


BLK = 2048
GRID = BATCH // BLK


def _mlp_body(x_ref, w1t_ref, b1_ref, wn_ref, ws_ref, bias_ref,
              out_n_ref, out_s_ref):
    x = x_ref[...]
    h = jnp.dot(x, w1t_ref[...], preferred_element_type=jnp.float32)
    h = jnp.maximum(h + b1_ref[...], 0.0)
    n = jnp.sum(h * wn_ref[...], axis=1, keepdims=True) + bias_ref[0]
    s = jnp.sum(h * ws_ref[...], axis=1, keepdims=True) + bias_ref[1]
    out_n_ref[...] = jax.nn.sigmoid(n).T
    out_s_ref[...] = jax.nn.sigmoid(s).T


def _mlp(x, w1t, b1r, wn, ws, bias2):
    return pl.pallas_call(
        _mlp_body,
        grid=(GRID,),
        in_specs=[
            pl.BlockSpec((BLK, 2 * DIM), lambda i: (i, 0)),
            pl.BlockSpec((2 * DIM, 2 * DIM), lambda i: (0, 0)),
            pl.BlockSpec((1, 2 * DIM), lambda i: (0, 0)),
            pl.BlockSpec((1, 2 * DIM), lambda i: (0, 0)),
            pl.BlockSpec((1, 2 * DIM), lambda i: (0, 0)),
            pl.BlockSpec(memory_space=pltpu.SMEM),
        ],
        out_specs=[
            pl.BlockSpec((1, BLK), lambda i: (0, i)),
            pl.BlockSpec((1, BLK), lambda i: (0, i)),
        ],
        out_shape=[
            jax.ShapeDtypeStruct((1, BATCH), jnp.float32),
            jax.ShapeDtypeStruct((1, BATCH), jnp.float32),
        ],
    )(x, w1t, b1r, wn, ws, bias2)


def kernel(inputs, emb, W1, b1, Wn, bn, Ws, bs):
    p = _pack(emb.T)
    x = _gather(inputs.T, p)
    bias2 = jnp.concatenate([bn, bs])
    out_n, out_s = _mlp(x, W1.T, b1.reshape(1, 2 * DIM), Wn, Ws, bias2)
    return (out_n.T, out_s.T)


# speedup vs baseline: 2.9502x; 1.0198x over previous
import functools

import jax
import jax.numpy as jnp
from jax import lax
from jax.experimental import pallas as pl
from jax.experimental.pallas import tpu as pltpu
from jax.experimental.pallas import tpu_sc as plsc

DIM = 64
BATCH = 16384
ROWS = 2 * BATCH
VOC = 1000000
NC = 2
NS = 16
NW = NC * NS
BPW = ROWS // NW
RPW = BATCH // NW
C = 128
NSTAGE = BPW // C


PK_VB = 32768
PK_SH = 15
PK_H = PK_VB // 2
PK_GRID = -(-VOC // PK_VB)
VOC2 = PK_GRID * PK_H


def _pack_body(xt_ref, p_ref):
    t = xt_ref[...]
    a = lax.slice(t, (0, 0), (DIM, PK_H))
    b = lax.slice(t, (0, PK_H), (DIM, PK_VB))
    p_ref[...] = jnp.concatenate([a, b], axis=0).T


def _pack(embT):
    return pl.pallas_call(
        _pack_body,
        grid=(PK_GRID,),
        in_specs=[pl.BlockSpec((DIM, PK_VB), lambda i: (0, i))],
        out_specs=pl.BlockSpec((PK_H, 2 * DIM), lambda i: (i, 0)),
        out_shape=jax.ShapeDtypeStruct((VOC2, 2 * DIM), jnp.float32),
    )(embT)


def _build_gather():
    mesh = plsc.VectorSubcoreMesh(core_axis_name="c", subcore_axis_name="s")

    @functools.partial(
        pl.kernel,
        mesh=mesh,
        compiler_params=pltpu.CompilerParams(needs_layout_passes=False),
        out_type=jax.ShapeDtypeStruct((BATCH, 2 * DIM), jnp.float32),
        scratch_types=[
            pltpu.VMEM((BPW,), jnp.int32),
            pltpu.VMEM((BPW,), jnp.int32),
            pltpu.VMEM((C, 2 * DIM), jnp.float32),
            pltpu.VMEM((C, 2 * DIM), jnp.float32),
            pltpu.VMEM((RPW, 2 * DIM), jnp.float32),
            pltpu.SemaphoreType.DMA,
            pltpu.SemaphoreType.DMA,
        ],
    )
    def gather_k(idx_hbm, p_hbm, out_hbm,
                 idx1_v, pk_v, buf0, buf1, out_v, sem0, sem1):
        wid = lax.axis_index("s") * NC + lax.axis_index("c")
        b0 = wid * RPW
        pltpu.sync_copy(idx_hbm.at[0, pl.ds(b0, RPW)], idx1_v.at[pl.ds(0, RPW)])
        pltpu.sync_copy(idx_hbm.at[1, pl.ds(b0, RPW)],
                        idx1_v.at[pl.ds(RPW, RPW)])
        for g in range(BPW // 16):
            v = idx1_v[pl.ds(16 * g, 16)]
            pk = (lax.shift_right_logical(v, PK_SH) * PK_H
                  + jnp.bitwise_and(v, PK_H - 1))
            pk_v[pl.ds(16 * g, 16)] = pk

        bufs = (buf0, buf1)
        sems = (sem0, sem1)

        def stage_copy(s, b):
            off = pl.multiple_of(s * C, 8)
            return pltpu.make_async_copy(
                p_hbm.at[pk_v.at[pl.ds(off, C)]], bufs[b], sems[b])

        stage_copy(0, 0).start()
        lane = lax.iota(jnp.int32, 16)

        def extract(s, buf):
            r0 = jnp.bitwise_and(s, 3) * C
            c0 = lax.shift_right_logical(s, 2) * 64

            def per_j(j):
                vv = plsc.load_gather(
                    idx1_v, [jnp.full((16,), s * C + j, jnp.int32)])
                hv = jnp.bitwise_and(
                    lax.shift_right_logical(vv, PK_SH - 1), 1) * 64
                jv = jnp.full((16,), j, jnp.int32)
                r = r0 + j
                for k in range(4):
                    x = plsc.load_gather(buf, [jv, hv + lane + 16 * k])
                    out_v[r, pl.ds(c0 + 16 * k, 16)] = x

            pl.loop(0, C)(per_j)

        def body(s0):
            for b in range(2):
                s = s0 + b

                @pl.when(s + 1 < NSTAGE)
                def _():
                    stage_copy(s + 1, 1 - b).start()

                stage_copy(s, b).wait()
                extract(s, bufs[b])

        pl.loop(0, NSTAGE, step=2)(body)
        pltpu.sync_copy(out_v, out_hbm.at[pl.ds(wid * RPW, RPW)])

    return gather_k


_gather = _build_gather()


BLK = 2048
GRID = BATCH // BLK


def _mlp_body(x_ref, w1t_ref, b1_ref, wn_ref, ws_ref, bias_ref,
              out_n_ref, out_s_ref):
    x = x_ref[...]
    h = jnp.dot(x, w1t_ref[...], preferred_element_type=jnp.float32)
    h = jnp.maximum(h + b1_ref[...], 0.0)
    n = jnp.sum(h * wn_ref[...], axis=1, keepdims=True) + bias_ref[0]
    s = jnp.sum(h * ws_ref[...], axis=1, keepdims=True) + bias_ref[1]
    out_n_ref[...] = jax.nn.sigmoid(n).T
    out_s_ref[...] = jax.nn.sigmoid(s).T


def _mlp(x, w1t, b1r, wn, ws, bias2):
    return pl.pallas_call(
        _mlp_body,
        grid=(GRID,),
        in_specs=[
            pl.BlockSpec((BLK, 2 * DIM), lambda i: (i, 0)),
            pl.BlockSpec((2 * DIM, 2 * DIM), lambda i: (0, 0)),
            pl.BlockSpec((1, 2 * DIM), lambda i: (0, 0)),
            pl.BlockSpec((1, 2 * DIM), lambda i: (0, 0)),
            pl.BlockSpec((1, 2 * DIM), lambda i: (0, 0)),
            pl.BlockSpec(memory_space=pltpu.SMEM),
        ],
        out_specs=[
            pl.BlockSpec((1, BLK), lambda i: (0, i)),
            pl.BlockSpec((1, BLK), lambda i: (0, i)),
        ],
        out_shape=[
            jax.ShapeDtypeStruct((1, BATCH), jnp.float32),
            jax.ShapeDtypeStruct((1, BATCH), jnp.float32),
        ],
    )(x, w1t, b1r, wn, ws, bias2)


def kernel(inputs, emb, W1, b1, Wn, bn, Ws, bs):
    p = _pack(emb.T)
    x = _gather(inputs.T, p)
    bias2 = jnp.concatenate([bn, bs])
    out_n, out_s = _mlp(x, W1.T, b1.reshape(1, 2 * DIM), Wn, Ws, bias2)
    return (out_n.T, out_s.T)
